# baseline (speedup 1.0000x reference)
import functools

import jax
import jax.numpy as jnp
from jax import lax
from jax.experimental import pallas as pl
from jax.experimental.pallas import tpu as pltpu
from jax.experimental.pallas import tpu_sc as plsc

N = 100000
D = 256
C = 1000
CP = 1024
K = 80
NCHUNKS = N // K
NW = 32
NITER = NCHUNKS // NW
ALPHA = 1.0
TAU_LAMBDA = 1.5

_f32 = jnp.float32
_i32 = jnp.int32

_GDN = lax.GatherDimensionNumbers(offset_dims=(), collapsed_slice_dims=(0,),
                                  start_index_map=(0,))


def _permute(v, idx):
    return lax.gather(v, idx[:, None], _GDN, (1,),
                      mode=lax.GatherScatterMode.PROMISE_IN_BOUNDS)


def _prep_body(text_ref, that_ref):
    t = text_ref[...]
    nrm = jnp.sqrt(jnp.sum(t * t, axis=-1, keepdims=True))
    that_ref[...] = (t / jnp.maximum(nrm, 1e-8)).reshape(2 * C, D // 2)


_prep = pl.pallas_call(
    _prep_body,
    out_shape=jax.ShapeDtypeStruct((2 * C, D // 2), _f32),
)


def _sc_body(g_hbm, lbl_hbm, that_hbm,
             cs_out, cnt_out, sd_out, sd2_out,
             lbl_v, ia, ib, g_lo, g_hi, a_v,
             cnt_v, sd_v, sd2_v, acc,
             sem_l, sem_gl, sem_gh, sem_aa, sem_ab):
    cid = lax.axis_index("c")
    sid = lax.axis_index("s")
    wid = sid * 2 + cid

    z16 = jnp.zeros((16,), _f32)

    for r in range(8):
        for cc in range(8):
            cnt_v[r, pl.ds(cc * 16, 16)] = z16
            sd_v[r, pl.ds(cc * 16, 16)] = z16
            sd2_v[r, pl.ds(cc * 16, 16)] = z16

    @pl.when(sid < 8)
    def _zero_acc():
        def _zrow(i, _):
            for u in range(8):
                a_v[0, i, pl.ds(u * 16, 16)] = z16
            return 0

        lax.fori_loop(0, 128, _zrow, 0)
        pltpu.sync_copy(a_v.at[0, pl.ds(0, 128)],
                        acc.at[pl.ds(sid * 256, 128)])
        pltpu.sync_copy(a_v.at[0, pl.ds(0, 128)],
                        acc.at[pl.ds(sid * 256 + 128, 128)])

    plsc.subcore_barrier()

    lanes = lax.iota(_i32, 16)
    onesf = jnp.ones((16,), _f32)

    def _issue(c, b):
        base = c * K
        pltpu.async_copy(lbl_hbm.at[pl.ds(base, K)], lbl_v.at[b],
                         sem_l).wait()
        for j in range(K // 16):
            l2 = lbl_v[b, pl.ds(j * 16, 16)] * 2
            ia[b, pl.ds(j * 16, 16)] = l2
            ib[b, pl.ds(j * 16, 16)] = l2 + 1
        pltpu.async_copy(g_hbm.at[pl.ds(base, K), pl.ds(0, 128)],
                         g_lo.at[b], sem_gl)
        pltpu.async_copy(g_hbm.at[pl.ds(base, K), pl.ds(128, 128)],
                         g_hi.at[b], sem_gh)
        pltpu.async_copy(that_hbm.at[ia.at[b]], a_v.at[b, pl.ds(0, K)],
                         sem_aa)
        pltpu.async_copy(that_hbm.at[ib.at[b]], a_v.at[b, pl.ds(K, K)],
                         sem_ab)

    def _wait_inputs(c, b):
        base = c * K
        pltpu.make_async_copy(g_hbm.at[pl.ds(base, K), pl.ds(0, 128)],
                              g_lo.at[b], sem_gl).wait()
        pltpu.make_async_copy(g_hbm.at[pl.ds(base, K), pl.ds(128, 128)],
                              g_hi.at[b], sem_gh).wait()
        pltpu.make_async_copy(that_hbm.at[ia.at[b]],
                              a_v.at[b, pl.ds(0, K)], sem_aa).wait()
        pltpu.make_async_copy(that_hbm.at[ib.at[b]],
                              a_v.at[b, pl.ds(K, K)], sem_ab).wait()

    def _compute(b):
        for gi in range(K // 16):
            labels_g = lbl_v[b, pl.ds(gi * 16, 16)]

            def _samp(t2, res):
                rd, rg = res
                for t in (t2 * 2, t2 * 2 + 1):
                    s = gi * 16 + t
                    da = [jnp.zeros((16,), _f32) for _ in range(4)]
                    ga = [jnp.zeros((16,), _f32) for _ in range(4)]
                    for u in range(8):
                        gv = g_lo[b, s, pl.ds(u * 16, 16)]
                        av = a_v[b, s, pl.ds(u * 16, 16)]
                        da[u % 4] = da[u % 4] + gv * av
                        ga[u % 4] = ga[u % 4] + gv * gv
                    for u in range(8):
                        gv = g_hi[b, s, pl.ds(u * 16, 16)]
                        av = a_v[b, K + s, pl.ds(u * 16, 16)]
                        da[u % 4] = da[u % 4] + gv * av
                        ga[u % 4] = ga[u % 4] + gv * gv
                    dv = (da[0] + da[1]) + (da[2] + da[3])
                    g2v = (ga[0] + ga[1]) + (ga[2] + ga[3])
                    for sh in (8, 4, 2, 1):
                        dv = dv + _permute(dv, lanes ^ sh)
                        g2v = g2v + _permute(g2v, lanes ^ sh)
                    m = lanes == t
                    rd = jnp.where(m, dv, rd)
                    rg = jnp.where(m, g2v, rg)
                return rd, rg

            dot, g2 = lax.fori_loop(0, 8, _samp,
                                    (jnp.zeros((16,), _f32),
                                     jnp.zeros((16,), _f32)))

            g2c = jnp.maximum(g2, 1e-16)
            bits = plsc.bitcast(g2c, _i32)
            y = plsc.bitcast(jnp.int32(0x5F3759DF) - (bits >> 1), _f32)
            for _ in range(3):
                y = y * (1.5 - 0.5 * g2c * y * y)
            dd = 1.0 - dot * y
            lhi = lax.shift_right_logical(labels_g, 7)
            llo = labels_g & 127
            plsc.addupdate_scatter(cnt_v, [lhi, llo], onesf)
            plsc.addupdate_scatter(sd_v, [lhi, llo], dd)
            plsc.addupdate_scatter(sd2_v, [lhi, llo], dd * dd)

    def _scatter(b):
        pltpu.sync_copy(g_lo.at[b], acc.at[ia.at[b]], add=True)
        pltpu.sync_copy(g_hi.at[b], acc.at[ib.at[b]], add=True)

    _issue(wid, 0)

    def _chunk(i, _):
        b = i & 1
        _wait_inputs(wid + i * 32, b)

        @pl.when(i < NITER - 1)
        def _next():
            _issue(wid + (i + 1) * 32, 1 - b)

        _compute(b)
        _scatter(b)
        return 0

    lax.fori_loop(0, NITER, _chunk, 0)

    @pl.when(wid < 2)
    def _tail():
        c = NITER * 32 + wid
        _issue(c, 0)
        _wait_inputs(c, 0)
        _compute(0)
        _scatter(0)

    plsc.subcore_barrier()

    pltpu.sync_copy(cnt_v, cnt_out.at[wid])
    pltpu.sync_copy(sd_v, sd_out.at[wid])
    pltpu.sync_copy(sd2_v, sd2_out.at[wid])

    @pl.when(sid < 8)
    def _flush_acc():
        pltpu.sync_copy(acc.at[pl.ds(sid * 256, 256)],
                        cs_out.at[cid, pl.ds(sid * 256, 256)])


_sc = functools.partial(
    pl.kernel,
    out_type=(
        jax.ShapeDtypeStruct((2, 2 * CP, D // 2), _f32),
        jax.ShapeDtypeStruct((NW, 8, 128), _f32),
        jax.ShapeDtypeStruct((NW, 8, 128), _f32),
        jax.ShapeDtypeStruct((NW, 8, 128), _f32),
    ),
    mesh=plsc.VectorSubcoreMesh(core_axis_name="c", subcore_axis_name="s",
                                num_cores=2, num_subcores=16),
    compiler_params=pltpu.CompilerParams(use_tc_tiling_on_sc=True,
                                         needs_layout_passes=False),
    scratch_types=[
        pltpu.VMEM((2, K), _i32),
        pltpu.VMEM((2, K), _i32),
        pltpu.VMEM((2, K), _i32),
        pltpu.VMEM((2, K, 128), _f32),
        pltpu.VMEM((2, K, 128), _f32),
        pltpu.VMEM((2, 2 * K, 128), _f32),
        pltpu.VMEM((8, 128), _f32),
        pltpu.VMEM((8, 128), _f32),
        pltpu.VMEM((8, 128), _f32),
        pltpu.MemorySpace.VMEM_SHARED((2 * CP, D // 2), _f32),
        pltpu.SemaphoreType.DMA,
        pltpu.SemaphoreType.DMA,
        pltpu.SemaphoreType.DMA,
        pltpu.SemaphoreType.DMA,
        pltpu.SemaphoreType.DMA,
    ],
)(_sc_body)


def _fin_body(cs_ref, cnt_ref, sd_ref, sd2_ref, text_ref, uni_ref, tau_ref):
    counts = jnp.sum(cnt_ref[...], axis=0).reshape(CP)[:C]
    sum_d = jnp.sum(sd_ref[...], axis=0).reshape(CP)[:C]
    sum_d2 = jnp.sum(sd2_ref[...], axis=0).reshape(CP)[:C]
    cs = (cs_ref[0] + cs_ref[1]).reshape(CP, D)[:C]

    mu = sum_d / jnp.maximum(counts, 1.0)
    var = (sum_d2 - counts * mu * mu) / jnp.maximum(counts - 1.0, 1.0)
    std = jnp.sqrt(jnp.maximum(var, 0.0))
    tau = jnp.where(counts > 0,
                    jnp.where(std > 0, mu + TAU_LAMBDA * std, mu + 0.1),
                    0.0)

    visual = cs / jnp.maximum(counts, 1.0)[:, None]
    vn = jnp.sqrt(jnp.sum(visual * visual, axis=-1, keepdims=True))
    visual = visual / jnp.maximum(vn, 1e-12)
    uni = text_ref[...] + ALPHA * visual
    un = jnp.sqrt(jnp.sum(uni * uni, axis=-1, keepdims=True))
    uni_ref[...] = uni / jnp.maximum(un, 1e-12)
    tau_ref[...] = tau


_fin = pl.pallas_call(
    _fin_body,
    out_shape=(
        jax.ShapeDtypeStruct((C, D), _f32),
        jax.ShapeDtypeStruct((C,), _f32),
    ),
)


def kernel(support_global, support_labels, support_patches,
           support_patches_labels, text_features):
    del support_patches, support_patches_labels
    labels = support_labels.astype(_i32)
    that = _prep(text_features)
    cs, cnt, sd, sd2 = _sc(support_global, labels, that)
    return _fin(cs, cnt, sd, sd2, text_features)


# speedup vs baseline: 1.0632x; 1.0632x over previous
import functools

import jax
import jax.numpy as jnp
from jax import lax
from jax.experimental import pallas as pl
from jax.experimental.pallas import tpu as pltpu
from jax.experimental.pallas import tpu_sc as plsc

N = 100000
D = 256
C = 1000
CP = 1024
K = 80
NCHUNKS = N // K
NW = 32
NITER = NCHUNKS // NW
ALPHA = 1.0
TAU_LAMBDA = 1.5

_f32 = jnp.float32
_i32 = jnp.int32

_GDN = lax.GatherDimensionNumbers(offset_dims=(), collapsed_slice_dims=(0,),
                                  start_index_map=(0,))


def _permute(v, idx):
    return lax.gather(v, idx[:, None], _GDN, (1,),
                      mode=lax.GatherScatterMode.PROMISE_IN_BOUNDS)


def _prep_body(text_ref, that_ref):
    t = text_ref[...]
    nrm = jnp.sqrt(jnp.sum(t * t, axis=-1, keepdims=True))
    that_ref[...] = (t / jnp.maximum(nrm, 1e-8)).reshape(2 * C, D // 2)


_prep = pl.pallas_call(
    _prep_body,
    out_shape=jax.ShapeDtypeStruct((2 * C, D // 2), _f32),
)


def _sc_body(g_hbm, lbl_hbm, that_hbm,
             cs_out, cnt_out, sd_out, sd2_out,
             lbl_v, ia, ib, g_lo, g_hi, a_v,
             cnt_v, sd_v, sd2_v, acc,
             sem_l, sem_gl, sem_gh, sem_aa, sem_ab):
    cid = lax.axis_index("c")
    sid = lax.axis_index("s")
    wid = sid * 2 + cid

    z16 = jnp.zeros((16,), _f32)

    for r in range(8):
        for cc in range(8):
            cnt_v[r, pl.ds(cc * 16, 16)] = z16
            sd_v[r, pl.ds(cc * 16, 16)] = z16
            sd2_v[r, pl.ds(cc * 16, 16)] = z16

    @pl.when(sid < 8)
    def _zero_acc():
        def _zrow(i, _):
            for u in range(8):
                a_v[0, i, pl.ds(u * 16, 16)] = z16
            return 0

        lax.fori_loop(0, 128, _zrow, 0)
        pltpu.sync_copy(a_v.at[0, pl.ds(0, 128)],
                        acc.at[pl.ds(sid * 256, 128)])
        pltpu.sync_copy(a_v.at[0, pl.ds(0, 128)],
                        acc.at[pl.ds(sid * 256 + 128, 128)])

    plsc.subcore_barrier()

    lanes = lax.iota(_i32, 16)
    onesf = jnp.ones((16,), _f32)

    def _issue(c, b):
        base = c * K
        pltpu.async_copy(lbl_hbm.at[pl.ds(base, K)], lbl_v.at[b],
                         sem_l).wait()
        for j in range(K // 16):
            l2 = lbl_v[b, pl.ds(j * 16, 16)] * 2
            ia[b, pl.ds(j * 16, 16)] = l2
            ib[b, pl.ds(j * 16, 16)] = l2 + 1
        pltpu.async_copy(g_hbm.at[pl.ds(base, K), pl.ds(0, 128)],
                         g_lo.at[b], sem_gl)
        pltpu.async_copy(g_hbm.at[pl.ds(base, K), pl.ds(128, 128)],
                         g_hi.at[b], sem_gh)
        pltpu.async_copy(that_hbm.at[ia.at[b]], a_v.at[b, pl.ds(0, K)],
                         sem_aa)
        pltpu.async_copy(that_hbm.at[ib.at[b]], a_v.at[b, pl.ds(K, K)],
                         sem_ab)

    def _wait_inputs(c, b):
        base = c * K
        pltpu.make_async_copy(g_hbm.at[pl.ds(base, K), pl.ds(0, 128)],
                              g_lo.at[b], sem_gl).wait()
        pltpu.make_async_copy(g_hbm.at[pl.ds(base, K), pl.ds(128, 128)],
                              g_hi.at[b], sem_gh).wait()
        pltpu.make_async_copy(that_hbm.at[ia.at[b]],
                              a_v.at[b, pl.ds(0, K)], sem_aa).wait()
        pltpu.make_async_copy(that_hbm.at[ib.at[b]],
                              a_v.at[b, pl.ds(K, K)], sem_ab).wait()

    def _compute(b):
        for gi in range(K // 16):
            labels_g = lbl_v[b, pl.ds(gi * 16, 16)]

            def _samp(t, res):
                rd, rg = res
                s = gi * 16 + t
                da = [jnp.zeros((16,), _f32) for _ in range(4)]
                ga = [jnp.zeros((16,), _f32) for _ in range(4)]
                for u in range(8):
                    gv = g_lo[b, s, pl.ds(u * 16, 16)]
                    av = a_v[b, s, pl.ds(u * 16, 16)]
                    da[u % 4] = da[u % 4] + gv * av
                    ga[u % 4] = ga[u % 4] + gv * gv
                for u in range(8):
                    gv = g_hi[b, s, pl.ds(u * 16, 16)]
                    av = a_v[b, K + s, pl.ds(u * 16, 16)]
                    da[u % 4] = da[u % 4] + gv * av
                    ga[u % 4] = ga[u % 4] + gv * gv
                dv = (da[0] + da[1]) + (da[2] + da[3])
                g2v = (ga[0] + ga[1]) + (ga[2] + ga[3])
                for sh in (8, 4, 2, 1):
                    dv = dv + _permute(dv, lanes ^ sh)
                    g2v = g2v + _permute(g2v, lanes ^ sh)
                m = lanes == t
                return jnp.where(m, dv, rd), jnp.where(m, g2v, rg)

            dot, g2 = lax.fori_loop(0, 16, _samp,
                                    (jnp.zeros((16,), _f32),
                                     jnp.zeros((16,), _f32)))

            g2c = jnp.maximum(g2, 1e-16)
            bits = plsc.bitcast(g2c, _i32)
            y = plsc.bitcast(jnp.int32(0x5F3759DF) - (bits >> 1), _f32)
            for _ in range(3):
                y = y * (1.5 - 0.5 * g2c * y * y)
            dd = 1.0 - dot * y
            lhi = lax.shift_right_logical(labels_g, 7)
            llo = labels_g & 127
            plsc.addupdate_scatter(cnt_v, [lhi, llo], onesf)
            plsc.addupdate_scatter(sd_v, [lhi, llo], dd)
            plsc.addupdate_scatter(sd2_v, [lhi, llo], dd * dd)

    def _scatter(b):
        pltpu.sync_copy(g_lo.at[b], acc.at[ia.at[b]], add=True)
        pltpu.sync_copy(g_hi.at[b], acc.at[ib.at[b]], add=True)

    _issue(wid, 0)

    def _chunk(i, _):
        b = i & 1
        _wait_inputs(wid + i * 32, b)

        @pl.when(i < NITER - 1)
        def _next():
            _issue(wid + (i + 1) * 32, 1 - b)

        _compute(b)
        _scatter(b)
        return 0

    lax.fori_loop(0, NITER, _chunk, 0)

    @pl.when(wid < 2)
    def _tail():
        c = NITER * 32 + wid
        _issue(c, 0)
        _wait_inputs(c, 0)
        _compute(0)
        _scatter(0)

    plsc.subcore_barrier()

    pltpu.sync_copy(cnt_v, cnt_out.at[wid])
    pltpu.sync_copy(sd_v, sd_out.at[wid])
    pltpu.sync_copy(sd2_v, sd2_out.at[wid])

    @pl.when(sid < 8)
    def _flush_acc():
        pltpu.sync_copy(acc.at[pl.ds(sid * 256, 256)],
                        cs_out.at[cid, pl.ds(sid * 256, 256)])


_sc = functools.partial(
    pl.kernel,
    out_type=(
        jax.ShapeDtypeStruct((2, 2 * CP, D // 2), _f32),
        jax.ShapeDtypeStruct((NW, 8, 128), _f32),
        jax.ShapeDtypeStruct((NW, 8, 128), _f32),
        jax.ShapeDtypeStruct((NW, 8, 128), _f32),
    ),
    mesh=plsc.VectorSubcoreMesh(core_axis_name="c", subcore_axis_name="s",
                                num_cores=2, num_subcores=16),
    compiler_params=pltpu.CompilerParams(use_tc_tiling_on_sc=True,
                                         needs_layout_passes=False),
    scratch_types=[
        pltpu.VMEM((2, K), _i32),
        pltpu.VMEM((2, K), _i32),
        pltpu.VMEM((2, K), _i32),
        pltpu.VMEM((2, K, 128), _f32),
        pltpu.VMEM((2, K, 128), _f32),
        pltpu.VMEM((2, 2 * K, 128), _f32),
        pltpu.VMEM((8, 128), _f32),
        pltpu.VMEM((8, 128), _f32),
        pltpu.VMEM((8, 128), _f32),
        pltpu.MemorySpace.VMEM_SHARED((2 * CP, D // 2), _f32),
        pltpu.SemaphoreType.DMA,
        pltpu.SemaphoreType.DMA,
        pltpu.SemaphoreType.DMA,
        pltpu.SemaphoreType.DMA,
        pltpu.SemaphoreType.DMA,
    ],
)(_sc_body)


def _fin_body(cs_ref, cnt_ref, sd_ref, sd2_ref, text_ref, uni_ref, tau_ref):
    counts = jnp.sum(cnt_ref[...], axis=0).reshape(CP)[:C]
    sum_d = jnp.sum(sd_ref[...], axis=0).reshape(CP)[:C]
    sum_d2 = jnp.sum(sd2_ref[...], axis=0).reshape(CP)[:C]
    cs = (cs_ref[0] + cs_ref[1]).reshape(CP, D)[:C]

    mu = sum_d / jnp.maximum(counts, 1.0)
    var = (sum_d2 - counts * mu * mu) / jnp.maximum(counts - 1.0, 1.0)
    std = jnp.sqrt(jnp.maximum(var, 0.0))
    tau = jnp.where(counts > 0,
                    jnp.where(std > 0, mu + TAU_LAMBDA * std, mu + 0.1),
                    0.0)

    visual = cs / jnp.maximum(counts, 1.0)[:, None]
    vn = jnp.sqrt(jnp.sum(visual * visual, axis=-1, keepdims=True))
    visual = visual / jnp.maximum(vn, 1e-12)
    uni = text_ref[...] + ALPHA * visual
    un = jnp.sqrt(jnp.sum(uni * uni, axis=-1, keepdims=True))
    uni_ref[...] = uni / jnp.maximum(un, 1e-12)
    tau_ref[...] = tau


_fin = pl.pallas_call(
    _fin_body,
    out_shape=(
        jax.ShapeDtypeStruct((C, D), _f32),
        jax.ShapeDtypeStruct((C,), _f32),
    ),
)


def kernel(support_global, support_labels, support_patches,
           support_patches_labels, text_features):
    del support_patches, support_patches_labels
    labels = support_labels.astype(_i32)
    that = _prep(text_features)
    cs, cnt, sd, sd2 = _sc(support_global, labels, that)
    return _fin(cs, cnt, sd, sd2, text_features)


# speedup vs baseline: 1.0983x; 1.0330x over previous
import functools

import jax
import jax.numpy as jnp
from jax import lax
from jax.experimental import pallas as pl
from jax.experimental.pallas import tpu as pltpu
from jax.experimental.pallas import tpu_sc as plsc

N = 100000
D = 256
C = 1000
CP = 1024
K = 80
NCHUNKS = N // K
NW = 32
NITER = NCHUNKS // NW
ALPHA = 1.0
TAU_LAMBDA = 1.5

_f32 = jnp.float32
_i32 = jnp.int32

_GDN = lax.GatherDimensionNumbers(offset_dims=(), collapsed_slice_dims=(0,),
                                  start_index_map=(0,))


def _permute(v, idx):
    return lax.gather(v, idx[:, None], _GDN, (1,),
                      mode=lax.GatherScatterMode.PROMISE_IN_BOUNDS)


def _prep_body(text_ref, that_ref):
    t = text_ref[...]
    nrm = jnp.sqrt(jnp.sum(t * t, axis=-1, keepdims=True))
    that_ref[...] = (t / jnp.maximum(nrm, 1e-8)).reshape(2 * C, D // 2)


_prep = pl.pallas_call(
    _prep_body,
    out_shape=jax.ShapeDtypeStruct((2 * C, D // 2), _f32),
)


def _sc_body(g_hbm, lbl_hbm, that_hbm,
             cs_out, cnt_out, sd_out, sd2_out,
             lbl_v, ia, ib, g_lo, g_hi, a_v,
             cnt_v, sd_v, sd2_v, acc,
             sem_l, sem_gl, sem_gh, sem_aa, sem_ab):
    cid = lax.axis_index("c")
    sid = lax.axis_index("s")
    wid = sid * 2 + cid

    z16 = jnp.zeros((16,), _f32)

    for r in range(8):
        for cc in range(8):
            cnt_v[r, pl.ds(cc * 16, 16)] = z16
            sd_v[r, pl.ds(cc * 16, 16)] = z16
            sd2_v[r, pl.ds(cc * 16, 16)] = z16

    @pl.when(sid < 8)
    def _zero_acc():
        def _zrow(i, _):
            for u in range(8):
                a_v[0, i, pl.ds(u * 16, 16)] = z16
            return 0

        lax.fori_loop(0, 128, _zrow, 0)
        pltpu.sync_copy(a_v.at[0, pl.ds(0, 128)],
                        acc.at[pl.ds(sid * 256, 128)])
        pltpu.sync_copy(a_v.at[0, pl.ds(0, 128)],
                        acc.at[pl.ds(sid * 256 + 128, 128)])

    plsc.subcore_barrier()

    lanes = lax.iota(_i32, 16)
    onesf = jnp.ones((16,), _f32)

    def _fetch_labels(c, b):
        pltpu.async_copy(lbl_hbm.at[pl.ds(c * K, K)], lbl_v.at[b], sem_l)

    def _issue(c, b):
        base = c * K
        pltpu.make_async_copy(lbl_hbm.at[pl.ds(base, K)], lbl_v.at[b],
                              sem_l).wait()
        for j in range(K // 16):
            l2 = lbl_v[b, pl.ds(j * 16, 16)] * 2
            ia[b, pl.ds(j * 16, 16)] = l2
            ib[b, pl.ds(j * 16, 16)] = l2 + 1
        pltpu.async_copy(g_hbm.at[pl.ds(base, K), pl.ds(0, 128)],
                         g_lo.at[b], sem_gl)
        pltpu.async_copy(g_hbm.at[pl.ds(base, K), pl.ds(128, 128)],
                         g_hi.at[b], sem_gh)
        pltpu.async_copy(that_hbm.at[ia.at[b]], a_v.at[b, pl.ds(0, K)],
                         sem_aa)
        pltpu.async_copy(that_hbm.at[ib.at[b]], a_v.at[b, pl.ds(K, K)],
                         sem_ab)

    def _wait_inputs(c, b):
        base = c * K
        pltpu.make_async_copy(g_hbm.at[pl.ds(base, K), pl.ds(0, 128)],
                              g_lo.at[b], sem_gl).wait()
        pltpu.make_async_copy(g_hbm.at[pl.ds(base, K), pl.ds(128, 128)],
                              g_hi.at[b], sem_gh).wait()
        pltpu.make_async_copy(that_hbm.at[ia.at[b]],
                              a_v.at[b, pl.ds(0, K)], sem_aa).wait()
        pltpu.make_async_copy(that_hbm.at[ib.at[b]],
                              a_v.at[b, pl.ds(K, K)], sem_ab).wait()

    def _compute(b):
        for gi in range(K // 16):
            labels_g = lbl_v[b, pl.ds(gi * 16, 16)]

            def _samp(t, res):
                rd, rg = res
                s = gi * 16 + t
                da = [jnp.zeros((16,), _f32) for _ in range(4)]
                ga = [jnp.zeros((16,), _f32) for _ in range(4)]
                for u in range(8):
                    gv = g_lo[b, s, pl.ds(u * 16, 16)]
                    av = a_v[b, s, pl.ds(u * 16, 16)]
                    da[u % 4] = da[u % 4] + gv * av
                    ga[u % 4] = ga[u % 4] + gv * gv
                for u in range(8):
                    gv = g_hi[b, s, pl.ds(u * 16, 16)]
                    av = a_v[b, K + s, pl.ds(u * 16, 16)]
                    da[u % 4] = da[u % 4] + gv * av
                    ga[u % 4] = ga[u % 4] + gv * gv
                dv = (da[0] + da[1]) + (da[2] + da[3])
                g2v = (ga[0] + ga[1]) + (ga[2] + ga[3])
                for sh in (8, 4, 2, 1):
                    dv = dv + _permute(dv, lanes ^ sh)
                    g2v = g2v + _permute(g2v, lanes ^ sh)
                m = lanes == t
                return jnp.where(m, dv, rd), jnp.where(m, g2v, rg)

            dot, g2 = lax.fori_loop(0, 16, _samp,
                                    (jnp.zeros((16,), _f32),
                                     jnp.zeros((16,), _f32)))

            g2c = jnp.maximum(g2, 1e-16)
            bits = plsc.bitcast(g2c, _i32)
            y = plsc.bitcast(jnp.int32(0x5F3759DF) - (bits >> 1), _f32)
            for _ in range(3):
                y = y * (1.5 - 0.5 * g2c * y * y)
            dd = 1.0 - dot * y
            lhi = lax.shift_right_logical(labels_g, 7)
            llo = labels_g & 127
            plsc.addupdate_scatter(cnt_v, [lhi, llo], onesf)
            plsc.addupdate_scatter(sd_v, [lhi, llo], dd)
            plsc.addupdate_scatter(sd2_v, [lhi, llo], dd * dd)

    def _scatter(b):
        pltpu.sync_copy(g_lo.at[b], acc.at[ia.at[b]], add=True)
        pltpu.sync_copy(g_hi.at[b], acc.at[ib.at[b]], add=True)

    _fetch_labels(wid, 0)
    _issue(wid, 0)
    _fetch_labels(wid + 32, 1)

    def _chunk(i, _):
        b = i & 1
        _wait_inputs(wid + i * 32, b)

        @pl.when(i < NITER - 1)
        def _next():
            _issue(wid + (i + 1) * 32, 1 - b)

        _compute(b)
        _scatter(b)

        @pl.when(i < NITER - 2)
        def _pref():
            _fetch_labels(wid + (i + 2) * 32, b)

        return 0

    lax.fori_loop(0, NITER, _chunk, 0)

    @pl.when(wid < 2)
    def _tail():
        c = NITER * 32 + wid
        _fetch_labels(c, 0)
        _issue(c, 0)
        _wait_inputs(c, 0)
        _compute(0)
        _scatter(0)

    plsc.subcore_barrier()

    pltpu.sync_copy(cnt_v, cnt_out.at[wid])
    pltpu.sync_copy(sd_v, sd_out.at[wid])
    pltpu.sync_copy(sd2_v, sd2_out.at[wid])

    @pl.when(sid < 8)
    def _flush_acc():
        pltpu.sync_copy(acc.at[pl.ds(sid * 256, 256)],
                        cs_out.at[cid, pl.ds(sid * 256, 256)])


_sc = functools.partial(
    pl.kernel,
    out_type=(
        jax.ShapeDtypeStruct((2, 2 * CP, D // 2), _f32),
        jax.ShapeDtypeStruct((NW, 8, 128), _f32),
        jax.ShapeDtypeStruct((NW, 8, 128), _f32),
        jax.ShapeDtypeStruct((NW, 8, 128), _f32),
    ),
    mesh=plsc.VectorSubcoreMesh(core_axis_name="c", subcore_axis_name="s",
                                num_cores=2, num_subcores=16),
    compiler_params=pltpu.CompilerParams(use_tc_tiling_on_sc=True,
                                         needs_layout_passes=False),
    scratch_types=[
        pltpu.VMEM((2, K), _i32),
        pltpu.VMEM((2, K), _i32),
        pltpu.VMEM((2, K), _i32),
        pltpu.VMEM((2, K, 128), _f32),
        pltpu.VMEM((2, K, 128), _f32),
        pltpu.VMEM((2, 2 * K, 128), _f32),
        pltpu.VMEM((8, 128), _f32),
        pltpu.VMEM((8, 128), _f32),
        pltpu.VMEM((8, 128), _f32),
        pltpu.MemorySpace.VMEM_SHARED((2 * CP, D // 2), _f32),
        pltpu.SemaphoreType.DMA,
        pltpu.SemaphoreType.DMA,
        pltpu.SemaphoreType.DMA,
        pltpu.SemaphoreType.DMA,
        pltpu.SemaphoreType.DMA,
    ],
)(_sc_body)


def _fin_body(cs_ref, cnt_ref, sd_ref, sd2_ref, text_ref, uni_ref, tau_ref):
    counts = jnp.sum(cnt_ref[...], axis=0).reshape(CP)[:C]
    sum_d = jnp.sum(sd_ref[...], axis=0).reshape(CP)[:C]
    sum_d2 = jnp.sum(sd2_ref[...], axis=0).reshape(CP)[:C]
    cs = (cs_ref[0] + cs_ref[1]).reshape(CP, D)[:C]

    mu = sum_d / jnp.maximum(counts, 1.0)
    var = (sum_d2 - counts * mu * mu) / jnp.maximum(counts - 1.0, 1.0)
    std = jnp.sqrt(jnp.maximum(var, 0.0))
    tau = jnp.where(counts > 0,
                    jnp.where(std > 0, mu + TAU_LAMBDA * std, mu + 0.1),
                    0.0)

    visual = cs / jnp.maximum(counts, 1.0)[:, None]
    vn = jnp.sqrt(jnp.sum(visual * visual, axis=-1, keepdims=True))
    visual = visual / jnp.maximum(vn, 1e-12)
    uni = text_ref[...] + ALPHA * visual
    un = jnp.sqrt(jnp.sum(uni * uni, axis=-1, keepdims=True))
    uni_ref[...] = uni / jnp.maximum(un, 1e-12)
    tau_ref[...] = tau


_fin = pl.pallas_call(
    _fin_body,
    out_shape=(
        jax.ShapeDtypeStruct((C, D), _f32),
        jax.ShapeDtypeStruct((C,), _f32),
    ),
)


def kernel(support_global, support_labels, support_patches,
           support_patches_labels, text_features):
    del support_patches, support_patches_labels
    labels = support_labels.astype(_i32)
    that = _prep(text_features)
    cs, cnt, sd, sd2 = _sc(support_global, labels, that)
    return _fin(cs, cnt, sd, sd2, text_features)
